# baseline (speedup 1.0000x reference)
import functools

import jax
import jax.numpy as jnp
import numpy as np
from jax.experimental import pallas as pl
from jax.experimental.pallas import tpu as pltpu

B, C, H, W = 4, 96, 56, 56
R = 2
K_DPC = 5
TOPK = 5
N = H * W
NP = (H // R) * (W // R)
M = NP // 4
C2, C4 = 2 * C, 4 * C

_PREC_REF = None
def _DOTR():
    return dict(precision=_PREC_REF, preferred_element_type=jnp.float32)
_DOT = dict(precision=jax.lax.Precision.HIGHEST,
            preferred_element_type=jnp.float32)
_DOTG = dict(precision=None, preferred_element_type=jnp.float32)


def _gelu(t):
    return jax.nn.gelu(t, approximate=True)


def _hg_kernel(xT_ref, x_ref, fc1w_ref, fc1b_ref,
               ffn1w_ref, ffn1b_ref, ffn2w_ref, ffn2b_ref,
               nnwa_ref, nnwb_ref, nnb_ref, fc2w_ref, fc2b_ref,
               out_ref):
    xT = xT_ref[0]
    x = x_ref[0]

    xfT = jax.lax.dot_general(xT, fc1w_ref[...],
                              (((1,), (0,)), ((), ())), **_DOTR()) + fc1b_ref[...]

    x4 = xfT.reshape(H // R, R, W // R, R, C)
    feats = ((x4[:, 0, :, 0] + x4[:, 0, :, 1])
             + (x4[:, 1, :, 0] + x4[:, 1, :, 1])).reshape(NP, C) * 0.25

    ny = jnp.sum(feats * feats, axis=-1, keepdims=True)
    g = jax.lax.dot_general(feats, feats,
                            (((1,), (1,)), ((), ())), **_DOTR())
    d2 = ny + jnp.transpose(ny) - 2.0 * g

    cols_np = jax.lax.broadcasted_iota(jnp.int32, (NP, NP), 1)
    cur = d2
    acc = jnp.zeros((NP, 1), jnp.float32)
    rem = jnp.full((NP, 1), jnp.float32(K_DPC))
    for _ in range(K_DPC):
        mn = jnp.min(cur, axis=-1, keepdims=True)
        eq = cur == mn
        cnt = jnp.sum(eq.astype(jnp.float32), axis=-1, keepdims=True)
        take = jnp.minimum(cnt, rem)
        acc = acc + mn * take
        rem = rem - take
        cur = jnp.where(eq, jnp.float32(3e38), cur)
    density = jnp.exp(-(acc * (1.0 / K_DPC)))

    higher = jnp.transpose(density) > density
    mdm = jnp.min(jnp.where(higher, d2, jnp.float32(1e10)), axis=-1, keepdims=True)
    rowmax = jnp.max(d2, axis=-1, keepdims=True)
    md = jnp.where(mdm >= 1e9, rowmax, mdm)
    score_i = density * md
    score_j = jnp.transpose(score_i)

    col_np = jax.lax.broadcasted_iota(jnp.int32, (1, NP), 1)
    row_np = jax.lax.broadcasted_iota(jnp.int32, (NP, 1), 0)
    beats = jnp.logical_or(score_j > score_i,
                           jnp.logical_and(score_j == score_i, col_np < row_np))
    rank = jnp.sum(beats.astype(jnp.float32), axis=-1, keepdims=True)

    rsel = jax.lax.broadcasted_iota(jnp.int32, (M, 1), 0)
    psel = (jnp.transpose(rank).astype(jnp.int32) == rsel).astype(jnp.float32)
    cent = jax.lax.dot_general(psel, feats,
                               (((1,), (0,)), ((), ())), **_DOT)

    nx = jnp.sum(xfT * xfT, axis=-1, keepdims=True)
    nc = jnp.sum(cent * cent, axis=-1, keepdims=True)
    gx = jax.lax.dot_general(xfT, cent,
                             (((1,), (1,)), ((), ())), **_DOTR())
    sim = 2.0 * gx - nx - jnp.transpose(nc)
    smax = jnp.max(sim, axis=-1, keepdims=True)
    e = jnp.exp(sim - smax)
    assign = e / jnp.sum(e, axis=-1, keepdims=True)

    num = jax.lax.dot_general(assign, xfT,
                              (((0,), (0,)), ((), ())), **_DOTR())
    den = jnp.sum(assign, axis=0)[:, None]
    agg = num / (den + 1e-6)

    t1 = _gelu(jax.lax.dot_general(agg, ffn1w_ref[...],
                                   (((1,), (1,)), ((), ())), **_DOTR()) + ffn1b_ref[...])
    t2 = jax.lax.dot_general(t1, ffn2w_ref[...],
                             (((1,), (1,)), ((), ())), **_DOTR()) + ffn2b_ref[...]
    agg2 = agg + t2

    cols_m = jax.lax.broadcasted_iota(jnp.int32, (1, M), 1)
    cur_a = assign
    xjmax = None
    for k in range(TOPK):
        jsel = jnp.argmax(cur_a, axis=-1)[:, None]
        onehot = (cols_m == jsel).astype(jnp.float32)
        row = jax.lax.dot_general(onehot, agg2,
                                  (((1,), (0,)), ((), ())), **_DOTG)
        xjmax = row if xjmax is None else jnp.maximum(xjmax, row)
        if k + 1 < TOPK:
            cur_a = jnp.where(onehot > 0.5, jnp.float32(-3e38), cur_a)
    xj = xjmax - xfT

    h = (jax.lax.dot_general(xfT, nnwa_ref[...],
                             (((1,), (1,)), ((), ())), **_DOTR())
         + jax.lax.dot_general(xj, nnwb_ref[...],
                               (((1,), (1,)), ((), ())), **_DOTR())
         + nnb_ref[...])
    h = _gelu(h)
    out = jax.lax.dot_general(fc2w_ref[...], h,
                              (((1,), (1,)), ((), ())), **_DOTR()) + fc2b_ref[...]
    out_ref[0] = out + x


def _impl(interpret, x, fc1_w, fc1_b, fc1_g, fc1_beta,
          ffn_w1, ffn_b1, ffn_g1, ffn_beta1,
          ffn_w2, ffn_b2, ffn_g2, ffn_beta2,
          nn_w, nn_b, nn_g, nn_beta,
          fc2_w, fc2_b, fc2_g, fc2_beta):
    f32 = jnp.float32
    xr = x.reshape(B, C, N)
    xT = xr.transpose(0, 2, 1)

    fc1w = (fc1_g[:, None] * fc1_w).T
    fc1b = (fc1_g * fc1_b + fc1_beta)[None, :]
    ffn1w = ffn_g1[:, None] * ffn_w1
    ffn1b = (ffn_g1 * ffn_b1 + ffn_beta1)[None, :]
    ffn2w = ffn_g2[:, None] * ffn_w2
    ffn2b = (ffn_g2 * ffn_b2 + ffn_beta2)[None, :]
    nnw = nn_g[:, None] * nn_w
    nnwa = nnw[:, 0::2]
    nnwb = nnw[:, 1::2]
    nnb = (nn_g * nn_b + nn_beta)[None, :]
    fc2w = fc2_g[:, None] * fc2_w
    fc2b = (fc2_g * fc2_b + fc2_beta)[:, None]

    full = lambda shp: pl.BlockSpec(shp, lambda b: (0,) * len(shp))
    out = pl.pallas_call(
        _hg_kernel,
        grid=(B,),
        in_specs=[
            pl.BlockSpec((1, N, C), lambda b: (b, 0, 0)),
            pl.BlockSpec((1, C, N), lambda b: (b, 0, 0)),
            full((C, C)), full((1, C)),
            full((C4, C)), full((1, C4)),
            full((C, C4)), full((1, C)),
            full((C2, C)), full((C2, C)), full((1, C2)),
            full((C, C2)), full((C, 1)),
        ],
        out_specs=pl.BlockSpec((1, C, N), lambda b: (b, 0, 0)),
        out_shape=jax.ShapeDtypeStruct((B, C, N), f32),
        compiler_params=pltpu.CompilerParams(
            dimension_semantics=("parallel",)),
        interpret=interpret,
    )(xT, xr, fc1w, fc1b, ffn1w, ffn1b, ffn2w, ffn2b,
      nnwa, nnwb, nnb, fc2w, fc2b)
    return out.reshape(B, C, H, W)


kernel = functools.partial(_impl, False)


# speedup vs baseline: 1.2404x; 1.2404x over previous
import functools

import jax
import jax.numpy as jnp
import numpy as np
from jax.experimental import pallas as pl
from jax.experimental.pallas import tpu as pltpu

B, C, H, W = 4, 96, 56, 56
R = 2
K_DPC = 5
TOPK = 5
N = H * W
NP = (H // R) * (W // R)
M = NP // 4
C2, C4 = 2 * C, 4 * C

_PREC_REF = None
def _DOTR():
    return dict(precision=_PREC_REF, preferred_element_type=jnp.float32)
_DOT = dict(precision=jax.lax.Precision.HIGHEST,
            preferred_element_type=jnp.float32)
_DOTG = dict(precision=None, preferred_element_type=jnp.float32)


def _gelu(t):
    return jax.nn.gelu(t, approximate=True)


def _hg_kernel(xT_ref, x_ref, fc1w_ref, fc1b_ref, fc1wc_ref, fc1bc_ref,
               ffn1w_ref, ffn1b_ref, ffn2w_ref, ffn2b_ref,
               nnwa_ref, nnwb_ref, nnb_ref, fc2w_ref, fc2b_ref,
               out_ref):
    xT = xT_ref[0]
    x = x_ref[0]

    xfT = jax.lax.dot_general(xT, fc1w_ref[...],
                              (((1,), (0,)), ((), ())), **_DOTR()) + fc1b_ref[...]
    xf = jax.lax.dot_general(fc1wc_ref[...], x,
                             (((1,), (0,)), ((), ())), **_DOTR()) + fc1bc_ref[...]

    x4 = xfT.reshape(H // R, R, W // R, R, C)
    feats = ((x4[:, 0, :, 0] + x4[:, 0, :, 1])
             + (x4[:, 1, :, 0] + x4[:, 1, :, 1])).reshape(NP, C) * 0.25

    ny = jnp.sum(feats * feats, axis=-1, keepdims=True)
    g = jax.lax.dot_general(feats, feats,
                            (((1,), (1,)), ((), ())), **_DOTR())
    d2 = ny + jnp.transpose(ny) - 2.0 * g

    cur = d2
    acc = jnp.zeros((NP, 1), jnp.float32)
    rem = jnp.full((NP, 1), jnp.float32(K_DPC))
    for _ in range(K_DPC):
        mn = jnp.min(cur, axis=-1, keepdims=True)
        eq = cur == mn
        cnt = jnp.sum(eq.astype(jnp.float32), axis=-1, keepdims=True)
        take = jnp.minimum(cnt, rem)
        acc = acc + mn * take
        rem = rem - take
        cur = jnp.where(eq, jnp.float32(3e38), cur)
    density = jnp.exp(-(acc * (1.0 / K_DPC)))

    higher = jnp.transpose(density) > density
    mdm = jnp.min(jnp.where(higher, d2, jnp.float32(1e10)), axis=-1, keepdims=True)
    rowmax = jnp.max(d2, axis=-1, keepdims=True)
    md = jnp.where(mdm >= 1e9, rowmax, mdm)
    score_i = density * md
    score_j = jnp.transpose(score_i)

    col_np = jax.lax.broadcasted_iota(jnp.int32, (1, NP), 1)
    row_np = jax.lax.broadcasted_iota(jnp.int32, (NP, 1), 0)
    beats = jnp.logical_or(score_j > score_i,
                           jnp.logical_and(score_j == score_i, col_np < row_np))
    rank = jnp.sum(beats.astype(jnp.float32), axis=-1, keepdims=True)

    rsel = jax.lax.broadcasted_iota(jnp.int32, (M, 1), 0)
    psel = (jnp.transpose(rank).astype(jnp.int32) == rsel).astype(jnp.float32)
    cent = jax.lax.dot_general(psel, feats,
                               (((1,), (0,)), ((), ())), **_DOT)

    nx = jnp.sum(xf * xf, axis=0, keepdims=True)
    nc = jnp.sum(cent * cent, axis=-1, keepdims=True)
    gx = jax.lax.dot_general(cent, xf,
                             (((1,), (0,)), ((), ())), **_DOTR())
    sim = 2.0 * gx - nx - nc
    smax = jnp.max(sim, axis=0, keepdims=True)
    e = jnp.exp(sim - smax)
    assign = e / jnp.sum(e, axis=0, keepdims=True)

    num = jax.lax.dot_general(xf, assign,
                              (((1,), (1,)), ((), ())), **_DOTR())
    ones_n = jnp.ones((1, N), jnp.float32)
    den = jax.lax.dot_general(ones_n, assign,
                              (((1,), (1,)), ((), ())), **_DOTR())
    agg = num / (den + 1e-6)

    t1 = _gelu(jax.lax.dot_general(ffn1w_ref[...], agg,
                                   (((1,), (0,)), ((), ())), **_DOTR()) + ffn1b_ref[...])
    t2 = jax.lax.dot_general(ffn2w_ref[...], t1,
                             (((1,), (0,)), ((), ())), **_DOTR()) + ffn2b_ref[...]
    agg2 = agg + t2

    rows_m = jax.lax.broadcasted_iota(jnp.int32, (M, N), 0)
    big = jnp.int32(2 ** 30)
    cur_a = assign
    xjmax = None
    for k in range(TOPK):
        mx = jnp.max(cur_a, axis=0, keepdims=True)
        wi = jnp.where(cur_a == mx, rows_m, big)
        mi = jnp.min(wi, axis=0, keepdims=True)
        hit = wi == mi
        onehot = hit.astype(jnp.float32)
        col = jax.lax.dot_general(agg2, onehot,
                                  (((1,), (0,)), ((), ())), **_DOTG)
        xjmax = col if xjmax is None else jnp.maximum(xjmax, col)
        if k + 1 < TOPK:
            cur_a = jnp.where(hit, jnp.float32(-3e38), cur_a)
    xj = xjmax - xf

    h = (jax.lax.dot_general(nnwa_ref[...], xf,
                             (((1,), (0,)), ((), ())), **_DOTR())
         + jax.lax.dot_general(nnwb_ref[...], xj,
                               (((1,), (0,)), ((), ())), **_DOTR())
         + nnb_ref[...])
    h = _gelu(h)
    out = jax.lax.dot_general(fc2w_ref[...], h,
                              (((1,), (0,)), ((), ())), **_DOTR()) + fc2b_ref[...]
    out_ref[0] = out + x


def _impl(interpret, x, fc1_w, fc1_b, fc1_g, fc1_beta,
          ffn_w1, ffn_b1, ffn_g1, ffn_beta1,
          ffn_w2, ffn_b2, ffn_g2, ffn_beta2,
          nn_w, nn_b, nn_g, nn_beta,
          fc2_w, fc2_b, fc2_g, fc2_beta):
    f32 = jnp.float32
    xr = x.reshape(B, C, N)
    xT = xr.transpose(0, 2, 1)

    fc1wc = fc1_g[:, None] * fc1_w
    fc1w = fc1wc.T
    fc1b = (fc1_g * fc1_b + fc1_beta)[None, :]
    fc1bc = (fc1_g * fc1_b + fc1_beta)[:, None]
    ffn1w = ffn_g1[:, None] * ffn_w1
    ffn1b = (ffn_g1 * ffn_b1 + ffn_beta1)[:, None]
    ffn2w = ffn_g2[:, None] * ffn_w2
    ffn2b = (ffn_g2 * ffn_b2 + ffn_beta2)[:, None]
    nnw = nn_g[:, None] * nn_w
    nnwa = nnw[:, 0::2]
    nnwb = nnw[:, 1::2]
    nnb = (nn_g * nn_b + nn_beta)[:, None]
    fc2w = fc2_g[:, None] * fc2_w
    fc2b = (fc2_g * fc2_b + fc2_beta)[:, None]

    full = lambda shp: pl.BlockSpec(shp, lambda b: (0,) * len(shp))
    out = pl.pallas_call(
        _hg_kernel,
        grid=(B,),
        in_specs=[
            pl.BlockSpec((1, N, C), lambda b: (b, 0, 0)),
            pl.BlockSpec((1, C, N), lambda b: (b, 0, 0)),
            full((C, C)), full((1, C)), full((C, C)), full((C, 1)),
            full((C4, C)), full((C4, 1)),
            full((C, C4)), full((C, 1)),
            full((C2, C)), full((C2, C)), full((C2, 1)),
            full((C, C2)), full((C, 1)),
        ],
        out_specs=pl.BlockSpec((1, C, N), lambda b: (b, 0, 0)),
        out_shape=jax.ShapeDtypeStruct((B, C, N), f32),
        compiler_params=pltpu.CompilerParams(
            dimension_semantics=("parallel",)),
        interpret=interpret,
    )(xT, xr, fc1w, fc1b, fc1wc, fc1bc, ffn1w, ffn1b, ffn2w, ffn2b,
      nnwa, nnwb, nnb, fc2w, fc2b)
    return out.reshape(B, C, H, W)


kernel = functools.partial(_impl, False)


# speedup vs baseline: 1.2450x; 1.0037x over previous
import functools

import jax
import jax.numpy as jnp
import numpy as np
from jax.experimental import pallas as pl
from jax.experimental.pallas import tpu as pltpu

B, C, H, W = 4, 96, 56, 56
R = 2
K_DPC = 5
TOPK = 5
N = H * W
NP = (H // R) * (W // R)
M = NP // 4
C2, C4 = 2 * C, 4 * C

_PREC_REF = None
def _DOTR():
    return dict(precision=_PREC_REF, preferred_element_type=jnp.float32)
_DOT = dict(precision=jax.lax.Precision.HIGHEST,
            preferred_element_type=jnp.float32)
_DOTG = dict(precision=None, preferred_element_type=jnp.float32)


def _gelu(t):
    return jax.nn.gelu(t, approximate=True)


def _hg_kernel(xT_ref, x_ref, fc1w_ref, fc1b_ref, fc1wc_ref, fc1bc_ref,
               ffn1w_ref, ffn1b_ref, ffn2w_ref, ffn2b_ref,
               nnwa_ref, nnwb_ref, nnb_ref, fc2w_ref, fc2b_ref,
               out_ref):
    xT = xT_ref[0]
    x = x_ref[0]

    xfT = jax.lax.dot_general(xT, fc1w_ref[...],
                              (((1,), (0,)), ((), ())), **_DOTR()) + fc1b_ref[...]
    xf = jax.lax.dot_general(fc1wc_ref[...], x,
                             (((1,), (0,)), ((), ())), **_DOTR()) + fc1bc_ref[...]

    x4 = xfT.reshape(H // R, R, W // R, R, C)
    feats = ((x4[:, 0, :, 0] + x4[:, 0, :, 1])
             + (x4[:, 1, :, 0] + x4[:, 1, :, 1])).reshape(NP, C) * 0.25

    ny = jnp.sum(feats * feats, axis=-1, keepdims=True)
    g = jax.lax.dot_general(feats, feats,
                            (((1,), (1,)), ((), ())), **_DOTR())
    d2 = ny + jnp.transpose(ny) - 2.0 * g

    cur = d2
    acc = jnp.zeros((1, NP), jnp.float32)
    rem = jnp.full((1, NP), jnp.float32(K_DPC))
    for _ in range(K_DPC):
        mn = jnp.min(cur, axis=0, keepdims=True)
        eq = cur == mn
        cnt = jnp.sum(eq.astype(jnp.float32), axis=0, keepdims=True)
        take = jnp.minimum(cnt, rem)
        acc = acc + mn * take
        rem = rem - take
        cur = jnp.where(eq, jnp.float32(3e38), cur)
    density = jnp.exp(-(acc * (1.0 / K_DPC)))
    density_c = jnp.transpose(density)

    higher = density_c > density
    mdm = jnp.min(jnp.where(higher, d2, jnp.float32(1e10)), axis=0, keepdims=True)
    colmax = jnp.max(d2, axis=0, keepdims=True)
    md = jnp.where(mdm >= 1e9, colmax, mdm)
    score_i = density * md
    score_j = jnp.transpose(score_i)

    col_np = jax.lax.broadcasted_iota(jnp.int32, (1, NP), 1)
    row_np = jax.lax.broadcasted_iota(jnp.int32, (NP, 1), 0)
    beats = jnp.logical_or(score_j > score_i,
                           jnp.logical_and(score_j == score_i, row_np < col_np))
    rank = jnp.sum(beats.astype(jnp.float32), axis=0, keepdims=True)

    rsel = jax.lax.broadcasted_iota(jnp.int32, (M, 1), 0)
    psel = (rank.astype(jnp.int32) == rsel).astype(jnp.float32)
    cent = jax.lax.dot_general(psel, feats,
                               (((1,), (0,)), ((), ())), **_DOT)

    nx = jnp.sum(xf * xf, axis=0, keepdims=True)
    nc = jnp.sum(cent * cent, axis=-1, keepdims=True)
    gx = jax.lax.dot_general(cent, xf,
                             (((1,), (0,)), ((), ())), **_DOTR())
    sim = 2.0 * gx - nx - nc
    smax = jnp.max(sim, axis=0, keepdims=True)
    e = jnp.exp(sim - smax)
    assign = e / jnp.sum(e, axis=0, keepdims=True)

    num = jax.lax.dot_general(xf, assign,
                              (((1,), (1,)), ((), ())), **_DOTR())
    ones_n = jnp.ones((1, N), jnp.float32)
    den = jax.lax.dot_general(ones_n, assign,
                              (((1,), (1,)), ((), ())), **_DOTR())
    agg = num / (den + 1e-6)

    t1 = _gelu(jax.lax.dot_general(ffn1w_ref[...], agg,
                                   (((1,), (0,)), ((), ())), **_DOTR()) + ffn1b_ref[...])
    t2 = jax.lax.dot_general(ffn2w_ref[...], t1,
                             (((1,), (0,)), ((), ())), **_DOTR()) + ffn2b_ref[...]
    agg2 = agg + t2

    rows_m = jax.lax.broadcasted_iota(jnp.int32, (M, N), 0)
    big = jnp.int32(2 ** 30)
    cur_a = assign
    xjmax = None
    for k in range(TOPK):
        mx = jnp.max(cur_a, axis=0, keepdims=True)
        wi = jnp.where(cur_a == mx, rows_m, big)
        mi = jnp.min(wi, axis=0, keepdims=True)
        hit = wi == mi
        onehot = hit.astype(jnp.float32)
        col = jax.lax.dot_general(agg2, onehot,
                                  (((1,), (0,)), ((), ())), **_DOTG)
        xjmax = col if xjmax is None else jnp.maximum(xjmax, col)
        if k + 1 < TOPK:
            cur_a = jnp.where(hit, jnp.float32(-3e38), cur_a)
    xj = xjmax - xf

    h = (jax.lax.dot_general(nnwa_ref[...], xf,
                             (((1,), (0,)), ((), ())), **_DOTR())
         + jax.lax.dot_general(nnwb_ref[...], xj,
                               (((1,), (0,)), ((), ())), **_DOTR())
         + nnb_ref[...])
    h = _gelu(h)
    out = jax.lax.dot_general(fc2w_ref[...], h,
                              (((1,), (0,)), ((), ())), **_DOTR()) + fc2b_ref[...]
    out_ref[0] = out + x


def _impl(interpret, x, fc1_w, fc1_b, fc1_g, fc1_beta,
          ffn_w1, ffn_b1, ffn_g1, ffn_beta1,
          ffn_w2, ffn_b2, ffn_g2, ffn_beta2,
          nn_w, nn_b, nn_g, nn_beta,
          fc2_w, fc2_b, fc2_g, fc2_beta):
    f32 = jnp.float32
    xr = x.reshape(B, C, N)
    xT = xr.transpose(0, 2, 1)

    fc1wc = fc1_g[:, None] * fc1_w
    fc1w = fc1wc.T
    fc1b = (fc1_g * fc1_b + fc1_beta)[None, :]
    fc1bc = (fc1_g * fc1_b + fc1_beta)[:, None]
    ffn1w = ffn_g1[:, None] * ffn_w1
    ffn1b = (ffn_g1 * ffn_b1 + ffn_beta1)[:, None]
    ffn2w = ffn_g2[:, None] * ffn_w2
    ffn2b = (ffn_g2 * ffn_b2 + ffn_beta2)[:, None]
    nnw = nn_g[:, None] * nn_w
    nnwa = nnw[:, 0::2]
    nnwb = nnw[:, 1::2]
    nnb = (nn_g * nn_b + nn_beta)[:, None]
    fc2w = fc2_g[:, None] * fc2_w
    fc2b = (fc2_g * fc2_b + fc2_beta)[:, None]

    full = lambda shp: pl.BlockSpec(shp, lambda b: (0,) * len(shp))
    out = pl.pallas_call(
        _hg_kernel,
        grid=(B,),
        in_specs=[
            pl.BlockSpec((1, N, C), lambda b: (b, 0, 0)),
            pl.BlockSpec((1, C, N), lambda b: (b, 0, 0)),
            full((C, C)), full((1, C)), full((C, C)), full((C, 1)),
            full((C4, C)), full((C4, 1)),
            full((C, C4)), full((C, 1)),
            full((C2, C)), full((C2, C)), full((C2, 1)),
            full((C, C2)), full((C, 1)),
        ],
        out_specs=pl.BlockSpec((1, C, N), lambda b: (b, 0, 0)),
        out_shape=jax.ShapeDtypeStruct((B, C, N), f32),
        compiler_params=pltpu.CompilerParams(
            dimension_semantics=("parallel",)),
        interpret=interpret,
    )(xT, xr, fc1w, fc1b, fc1wc, fc1bc, ffn1w, ffn1b, ffn2w, ffn2b,
      nnwa, nnwb, nnb, fc2w, fc2b)
    return out.reshape(B, C, H, W)


kernel = functools.partial(_impl, False)
